# baseline (speedup 1.0000x reference)
import functools

import jax
import jax.numpy as jnp
from jax import lax
from jax.experimental import pallas as pl
from jax.experimental.pallas import tpu as pltpu
from jax.experimental.pallas import tpu_sc as plsc

B = 4096
D = 64
L = 50
NU = 16
NI = 16
V = 100000
H1 = 512
H2 = 256

NC = 2
NS = 16
NW = NC * NS
CB = B // NW
LP = 64
NL = LP // 2
NSTREAM = CB * LP // 128


def _sc_gather_body(hist_idx, ucatT, icatT, ut0, ut1, ut2, it0, it1, it2,
                    htab,
                    hist_out, ucs_out, ics_out,
                    ihv, cidx, crow, cacc,
                    sidx0, sidx1, rows0, rows1, stage0, stage1,
                    semg0, semg1, semw0, semw1, semc):
    w = lax.axis_index("s") * NC + lax.axis_index("c")
    b0 = w * CB
    pltpu.sync_copy(hist_idx.at[pl.ds(b0, CB)], ihv)

    sidx = (sidx0, sidx1)
    rows = (rows0, rows1)
    stage = (stage0, stage1)
    semg = (semg0, semg1)
    semw = (semw0, semw1)
    zerosf16 = jnp.zeros((16,), jnp.float32)

    def zero_pads(st):
        def zp(k, carry):
            for h in (0, 1):
                for ch in range(8):
                    st[32 * h + 25 + k, pl.ds(16 * ch, 16)] = zerosf16
            return carry
        lax.fori_loop(0, 7, zp, 0)

    zero_pads(stage0)
    zero_pads(stage1)

    def build_idx(j, p):
        for h in (0, 1):
            r = 2 * j + h
            base = 50 * h
            for ch in range(3):
                sidx[p][pl.ds(base + 16 * ch, 16)] = ihv[r, pl.ds(16 * ch, 16)]
            sidx[p][pl.ds(base + 34, 16)] = ihv[r, pl.ds(34, 16)]

    def g_copy(p):
        return pltpu.make_async_copy(htab.at[sidx[p]], rows[p], semg[p])

    def w_copy(j, p):
        return pltpu.make_async_copy(
            stage[p], hist_out.at[pl.ds((w * NSTREAM + j) * 64, 64)],
            semw[p])

    def repack(p):
        def rp(it, carry):
            for h in (0, 1):
                for u in (0, 1):
                    src_r = 50 * h + 2 * it + u
                    dst_r = 32 * h + it
                    for ch in range(4):
                        stage[p][dst_r, pl.ds(D * u + 16 * ch, 16)] = (
                            rows[p][src_r, pl.ds(16 * ch, 16)])
            return carry
        lax.fori_loop(0, 25, rp, 0)

    build_idx(0, 0)
    g_copy(0).start()

    def step(it, carry):
        for b in (0, 1):
            j = 2 * it + b

            @pl.when(j + 1 < NSTREAM)
            def _():
                build_idx(j + 1, 1 - b)
                g_copy(1 - b).start()

            g_copy(b).wait()

            @pl.when(j >= 2)
            def _():
                w_copy(j - 2, b).wait()

            repack(b)
            w_copy(j, b).start()
        return carry

    lax.fori_loop(0, NSTREAM // 2, step, 0)
    w_copy(NSTREAM - 2, 0).wait()
    w_copy(NSTREAM - 1, 1).wait()

    for catT, tabs, out_ref in ((ucatT, (ut0, ut1, ut2), ucs_out),
                                (icatT, (it0, it1, it2), ics_out)):
        for t in range(3):
            pltpu.sync_copy(catT.at[t, pl.ds(b0, CB)], cidx)
            dst = cacc if t == 0 else crow
            pltpu.async_copy(tabs[t].at[cidx], dst, semc).wait()
            if t > 0:
                def add_step(r, carry):
                    for c in range(4):
                        sl = pl.ds(16 * c, 16)
                        cacc[r, sl] = cacc[r, sl] + crow[r, sl]
                    return carry
                lax.fori_loop(0, CB, add_step, 0)
        pltpu.sync_copy(cacc, out_ref.at[pl.ds(b0, CB)])


def _sc_gather(hist_idx, ucatT, icatT, ut0, ut1, ut2, it0, it1, it2, htab):
    mesh = plsc.VectorSubcoreMesh(core_axis_name="c", subcore_axis_name="s")
    f = functools.partial(
        pl.kernel,
        out_type=(
            jax.ShapeDtypeStruct((B * NL, 2 * D), jnp.float32),
            jax.ShapeDtypeStruct((B, D), jnp.float32),
            jax.ShapeDtypeStruct((B, D), jnp.float32),
        ),
        mesh=mesh,
        scratch_types=[
            pltpu.VMEM((CB, L), jnp.int32),
            pltpu.VMEM((CB,), jnp.int32),
            pltpu.VMEM((CB, D), jnp.float32),
            pltpu.VMEM((CB, D), jnp.float32),
            pltpu.VMEM((2 * L,), jnp.int32),
            pltpu.VMEM((2 * L,), jnp.int32),
            pltpu.VMEM((2 * L, D), jnp.float32),
            pltpu.VMEM((2 * L, D), jnp.float32),
            pltpu.VMEM((64, 2 * D), jnp.float32),
            pltpu.VMEM((64, 2 * D), jnp.float32),
            pltpu.SemaphoreType.DMA,
            pltpu.SemaphoreType.DMA,
            pltpu.SemaphoreType.DMA,
            pltpu.SemaphoreType.DMA,
            pltpu.SemaphoreType.DMA,
        ],
        compiler_params=pltpu.CompilerParams(use_tc_tiling_on_sc=False),
    )(_sc_gather_body)
    return f(hist_idx, ucatT, icatT, ut0, ut1, ut2, it0, it1, it2, htab)


R = 512


def _tc_body(un_ref, inum_ref, ucs_ref, ics_ref, hist_ref,
             Wun_ref, bun_ref, Wim_ref, bim_ref, wattn_ref,
             W1_ref, b1_ref, W2_ref, b2_ref, W3_ref, b3_ref, out_ref):
    f32 = jnp.float32
    ue = (jnp.dot(un_ref[...], Wun_ref[...], preferred_element_type=f32)
          + bun_ref[...] + ucs_ref[...])
    ie = (jnp.dot(inum_ref[...], Wim_ref[...], preferred_element_type=f32)
          + bim_ref[...] + ics_ref[...])
    hist = hist_ref[...].reshape(R, NL, 2 * D)
    qw = ie * wattn_ref[...]
    qw2 = jnp.concatenate([qw, qw], axis=1)
    prod = hist * qw2[:, None, :]
    lane = lax.broadcasted_iota(jnp.int32, (R, NL, 2 * D), 2)
    s_all = jnp.sum(prod, axis=2)
    s_e = jnp.sum(jnp.where(lane < D, prod, 0.0), axis=2)
    s_o = s_all - s_e
    t_e = jnp.tanh(s_e)
    t_o = jnp.tanh(s_o)
    k = lax.broadcasted_iota(jnp.int32, (R, NL), 1)
    e_e = jnp.where(k < L // 2, jnp.exp(t_e), 0.0)
    e_o = jnp.where(k < L // 2, jnp.exp(t_o), 0.0)
    z = jnp.sum(e_e + e_o, axis=1, keepdims=True)
    w_e = e_e / z
    w_o = e_o / z
    wfull = jnp.concatenate(
        [jnp.broadcast_to(w_e[:, :, None], (R, NL, D)),
         jnp.broadcast_to(w_o[:, :, None], (R, NL, D))], axis=2)
    att128 = jnp.sum(wfull * hist, axis=1)
    att = att128[:, :D] + att128[:, D:]
    comb = jnp.concatenate([ue, ie, att], axis=1)
    h = jnp.maximum(jnp.dot(comb, W1_ref[...], preferred_element_type=f32)
                    + b1_ref[...], 0.0)
    h = jnp.maximum(jnp.dot(h, W2_ref[...], preferred_element_type=f32)
                    + b2_ref[...], 0.0)
    logits = jnp.dot(h, W3_ref[...], preferred_element_type=f32) + b3_ref[...]
    out_ref[...] = jax.nn.sigmoid(logits)


def _tc_fused(user_num, item_num, ucs, ics, hist2,
              Wun, bun, Wim, bim, wattn, W1, b1, W2, b2, W3, b3):
    grid = (B // R,)
    full = lambda shape: pl.BlockSpec(shape, lambda i: (0,) * len(shape))
    return pl.pallas_call(
        _tc_body,
        grid=grid,
        in_specs=[
            pl.BlockSpec((R, NU), lambda i: (i, 0)),
            pl.BlockSpec((R, NI), lambda i: (i, 0)),
            pl.BlockSpec((R, D), lambda i: (i, 0)),
            pl.BlockSpec((R, D), lambda i: (i, 0)),
            pl.BlockSpec((R * NL, 2 * D), lambda i: (i, 0)),
            full((NU, D)), full((1, D)),
            full((NI, D)), full((1, D)), full((1, D)),
            full((3 * D, H1)), full((1, H1)),
            full((H1, H2)), full((1, H2)),
            full((H2, 1)), full((1, 1)),
        ],
        out_specs=pl.BlockSpec((R, 1), lambda i: (i, 0)),
        out_shape=jax.ShapeDtypeStruct((B, 1), jnp.float32),
    )(user_num, item_num, ucs, ics, hist2,
      Wun, bun, Wim, bim, wattn, W1, b1, W2, b2, W3, b3)


def kernel(user_num, item_num, user_cat, item_cat, history_items,
           Wun, bun, ut0, ut1, ut2, Wim, bim, it0, it1, it2,
           hist_tab, Wattn, W1, b1, W2, b2, W3, b3):
    hist2, ucs, ics = _sc_gather(
        history_items.astype(jnp.int32), user_cat.astype(jnp.int32).T,
        item_cat.astype(jnp.int32).T, ut0, ut1, ut2, it0, it1, it2, hist_tab)
    out = _tc_fused(user_num, item_num, ucs, ics,
                    hist2,
                    Wun, bun.reshape(1, D), Wim, bim.reshape(1, D),
                    Wattn.reshape(1, D), W1, b1.reshape(1, H1),
                    W2, b2.reshape(1, H2), W3, b3.reshape(1, 1))
    return out.reshape(B)
